# baseline (speedup 1.0000x reference)
import functools
import jax
import jax.numpy as jnp
from jax import lax
from jax.experimental import pallas as pl
from jax.experimental.pallas import tpu as pltpu
from jax.experimental.pallas import tpu_sc as plsc

_M, _N = 128, 32768
_NC, _NS, _L = 2, 16, 16
_NW = _NC * _NS
_RPW = _M // _NW
_CHUNK = 4096
_UNROLL = 8


def _sc_body(x_hbm, o_hbm, ib0, ib1, ob0, ob1, si0, si1, so0, so1):
    ibufs = (ib0, ib1)
    obufs = (ob0, ob1)
    sins = (si0, si1)
    souts = (so0, so1)
    wid = lax.axis_index("s") * _NC + lax.axis_index("c")
    r0 = wid * _RPW
    nch = _N // _CHUNK

    def cols(ci):
        return pl.ds(ci * _CHUNK, _CHUNK)

    def compute(ibuf, obuf, carries):
        def body(i, cs):
            cs = list(cs)
            base = i * (_L * _UNROLL)
            for u in range(_UNROLL):
                for r in range(_RPW):
                    v = ibuf[r, pl.ds(base + u * _L, _L)]
                    s = plsc.cumsum(v)
                    obuf[r, pl.ds(base + u * _L, _L)] = s + cs[r]
                    cs[r] = cs[r] + s[_L - 1]
            return tuple(cs)

        return lax.fori_loop(0, _CHUNK // (_L * _UNROLL), body, carries)

    def start_in(ci):
        return pltpu.async_copy(
            x_hbm.at[pl.ds(r0, _RPW), cols(ci)],
            ibufs[ci % 2], sins[ci % 2])

    def start_out(ci):
        return pltpu.async_copy(
            obufs[ci % 2],
            o_hbm.at[pl.ds(r0, _RPW), cols(ci)], souts[ci % 2])

    descs_in = {0: start_in(0), 1: start_in(1)}
    descs_out = {}
    carries = (jnp.float32(0),) * _RPW
    for ci in range(nch):
        b = ci % 2
        descs_in[ci].wait()
        if ci >= 2:
            descs_out[ci - 2].wait()
        carries = compute(ibufs[b], obufs[b], carries)
        descs_out[ci] = start_out(ci)
        if ci + 2 < nch:
            descs_in[ci + 2] = start_in(ci + 2)
    descs_out[nch - 2].wait()
    descs_out[nch - 1].wait()


def kernel(x):
    mesh = plsc.VectorSubcoreMesh(core_axis_name="c", subcore_axis_name="s")
    f = functools.partial(
        pl.kernel,
        mesh=mesh,
        out_type=jax.ShapeDtypeStruct((_M, _N), jnp.float32),
        scratch_types=[
            pltpu.VMEM((_RPW, _CHUNK), jnp.float32),
            pltpu.VMEM((_RPW, _CHUNK), jnp.float32),
            pltpu.VMEM((_RPW, _CHUNK), jnp.float32),
            pltpu.VMEM((_RPW, _CHUNK), jnp.float32),
            pltpu.SemaphoreType.DMA,
            pltpu.SemaphoreType.DMA,
            pltpu.SemaphoreType.DMA,
            pltpu.SemaphoreType.DMA,
        ],
        compiler_params=pltpu.CompilerParams(needs_layout_passes=False),
    )(_sc_body)
    return f(x)


# speedup vs baseline: 1.0100x; 1.0100x over previous
import functools
import jax
import jax.numpy as jnp
from jax import lax
from jax.experimental import pallas as pl
from jax.experimental.pallas import tpu as pltpu
from jax.experimental.pallas import tpu_sc as plsc

_M, _N = 128, 32768
_NC, _NS, _L = 2, 16, 16
_NW = _NC * _NS
_RPW = _M // _NW
_CHUNK = 8192
_UNROLL = 8


def _sc_body(x_hbm, o_hbm, buf0, buf1, si0, si1, so0, so1):
    bufs = (buf0, buf1)
    sins = (si0, si1)
    souts = (so0, so1)
    wid = lax.axis_index("s") * _NC + lax.axis_index("c")
    r0 = wid * _RPW
    nch = _N // _CHUNK

    def cols(ci):
        return pl.ds(ci * _CHUNK, _CHUNK)

    def compute(buf, carries):
        def body(i, cs):
            cs = list(cs)
            base = i * (_L * _UNROLL)
            for u in range(_UNROLL):
                for r in range(_RPW):
                    v = buf[r, pl.ds(base + u * _L, _L)]
                    s = plsc.cumsum(v)
                    buf[r, pl.ds(base + u * _L, _L)] = s + cs[r]
                    cs[r] = cs[r] + s[_L - 1]
            return tuple(cs)

        return lax.fori_loop(0, _CHUNK // (_L * _UNROLL), body, carries)

    descs_in = {}
    descs_out = {}
    descs_in[0] = pltpu.async_copy(
        x_hbm.at[pl.ds(r0, _RPW), cols(0)], bufs[0], sins[0])
    carries = (jnp.float32(0),) * _RPW
    for ci in range(nch):
        b = ci % 2
        descs_in[ci].wait()
        if ci + 1 < nch:
            if ci - 1 >= 0:
                descs_out[ci - 1].wait()
            descs_in[ci + 1] = pltpu.async_copy(
                x_hbm.at[pl.ds(r0, _RPW), cols(ci + 1)],
                bufs[1 - b], sins[1 - b])
        carries = compute(bufs[b], carries)
        descs_out[ci] = pltpu.async_copy(
            bufs[b], o_hbm.at[pl.ds(r0, _RPW), cols(ci)], souts[b])
    descs_out[nch - 2].wait()
    descs_out[nch - 1].wait()


def kernel(x):
    mesh = plsc.VectorSubcoreMesh(core_axis_name="c", subcore_axis_name="s")
    f = functools.partial(
        pl.kernel,
        mesh=mesh,
        out_type=jax.ShapeDtypeStruct((_M, _N), jnp.float32),
        scratch_types=[
            pltpu.VMEM((_RPW, _CHUNK), jnp.float32),
            pltpu.VMEM((_RPW, _CHUNK), jnp.float32),
            pltpu.SemaphoreType.DMA,
            pltpu.SemaphoreType.DMA,
            pltpu.SemaphoreType.DMA,
            pltpu.SemaphoreType.DMA,
        ],
        compiler_params=pltpu.CompilerParams(needs_layout_passes=False),
    )(_sc_body)
    return f(x)
